# Initial kernel scaffold; baseline (speedup 1.0000x reference)
import jax
import jax.numpy as jnp
from jax.experimental import pallas as pl


def kernel(x_time_series, x_graph, edge_index, edge_weight, params):
    raise NotImplementedError("write your pallas kernel here")


# speedup vs baseline: 1.7917x; 1.7917x over previous
import jax
import jax.numpy as jnp
from jax.experimental import pallas as pl

H = 128


def _gat(x, edge_index, ew, p, sum_w, cnt):
    n = x.shape[0]
    src, dst = edge_index[0], edge_index[1]
    loop_attr = sum_w / jnp.maximum(cnt, 1.0)
    h = x @ p["W"]
    a_src = h @ p["att_src"][0]
    a_dst = h @ p["att_dst"][0]
    c = jnp.sum(p["W_e"][0] * p["att_edge"][0])
    al_e = a_src[src] + a_dst[dst] + c * ew[:, 0]
    al_e = jnp.maximum(al_e, 0.2 * al_e)
    ex_e = jnp.exp(al_e)
    al_l = a_src + a_dst + c * loop_attr
    al_l = jnp.maximum(al_l, 0.2 * al_l)
    ex_l = jnp.exp(al_l)
    den = jax.ops.segment_sum(ex_e, dst, num_segments=n) + ex_l
    num = jax.ops.segment_sum(ex_e[:, None] * h[src], dst, num_segments=n) + ex_l[:, None] * h
    return num / (den[:, None] + 1e-16) + p["bias"]


def _final_relu_kernel(y_ref, o_ref):
    o_ref[...] = jnp.maximum(y_ref[...], 0.0)


def kernel(x_time_series, x_graph, edge_index, edge_weight, params):
    x = x_time_series[-1]
    n = x.shape[0]
    h = jnp.zeros((n, H), jnp.float32)
    p = params["gru"]
    for t in range(x.shape[1]):
        s = h @ p["W_attn"]
        attn = jax.nn.softmax(s, axis=0)
        gates = (x[:, t, :] * attn) @ p["W_ih"] + h @ p["W_hh"]
        r, u = gates[:, :H], gates[:, H:]
        r = jax.nn.sigmoid(r)
        u = jax.nn.sigmoid(u)
        h = u * h + (1 - u) * jnp.tanh(r * h)
    h1 = h
    ew = edge_weight
    dst = edge_index[1]
    cnt = jax.ops.segment_sum(jnp.ones_like(ew[:, 0]), dst, num_segments=n)
    sum_w = jax.ops.segment_sum(ew[:, 0], dst, num_segments=n)
    g = _gat(x_graph, edge_index, ew, params["gat1"], sum_w, cnt)
    g = jax.nn.relu(g)
    g = _gat(g, edge_index, ew, params["gat2"], sum_w, cnt)
    sc = H ** -0.5
    ca = params["ca"]

    def crossrep(q_in, kv):
        q = q_in @ ca["Wq"] + ca["bq"]
        k = kv @ ca["Wk"] + ca["bk"]
        v = kv @ ca["Wv"] + ca["bv"]
        aw = jax.nn.softmax((q @ k.T) * sc, axis=-1)
        return aw @ v

    rep1 = crossrep(h1, params["mhs1"])
    rep2 = crossrep(g, params["mhs2"])
    cat = jnp.concatenate([h1, g, rep1, rep2], axis=1)
    sa = cat @ params["sa"]["Wv"] + params["sa"]["bv"]
    y = _gat(sa, edge_index, ew, params["fgat1"], sum_w, cnt)
    y = jax.nn.relu(y)
    y = _gat(y, edge_index, ew, params["fgat2"], sum_w, cnt)
    y = y[:, 0]
    out = pl.pallas_call(
        _final_relu_kernel,
        out_shape=jax.ShapeDtypeStruct((n,), jnp.float32),
    )(y)
    return out


# speedup vs baseline: 37.5717x; 20.9695x over previous
import functools

import jax
import jax.numpy as jnp
from jax import lax
from jax.experimental import pallas as pl
from jax.experimental.pallas import tpu as pltpu
from jax.experimental.pallas import tpu_sc as plsc

H = 128
NC, NS, L = 2, 16, 16
NW = NC * NS
K = 80


def _make_sc_edge_pass(n, e, p, d):
    epw = e // NW
    nch = epw // K
    npw = n // NS
    ZR = 125
    nzc = npw // ZR
    nscale = (d + 1) + (L - 1)
    nscale //= L
    mesh = plsc.VectorSubcoreMesh(core_axis_name="c", subcore_axis_name="s")

    @functools.partial(
        pl.kernel,
        out_type=jax.ShapeDtypeStruct((NC, NS, n // NS, p), jnp.float32),
        mesh=mesh,
        compiler_params=pltpu.CompilerParams(needs_layout_passes=False,
                                             use_tc_tiling_on_sc=False),
        scratch_types=[
            pltpu.VMEM((nch, K), jnp.int32),
            pltpu.VMEM((nch, K), jnp.int32),
            pltpu.VMEM((nch, K), jnp.float32),
            pltpu.VMEM((n + L,), jnp.float32),
            pltpu.VMEM((n,), jnp.float32),
            pltpu.VMEM((K, p), jnp.float32),
            pltpu.VMEM((K, p), jnp.float32),
            pltpu.VMEM((ZR, p), jnp.float32),
            pltpu.VMEM_SHARED((n, p), jnp.float32),
            pltpu.SemaphoreType.DMA,
            pltpu.SemaphoreType.DMA,
        ],
    )
    def sc_kernel(src_hbm, dst_hbm, ew_hbm, asrc_hbm, adst_hbm, hpad_hbm,
                  out_hbm, src_v, dst_v, ew_v, asrc_v, adst_v, graw,
                  bscat, zbuf, acc, gsem, ssem):
        c_ax = lax.axis_index("c")
        s_ax = lax.axis_index("s")
        wid = c_ax * NS + s_ax
        pltpu.sync_copy(src_hbm.at[wid], src_v)
        pltpu.sync_copy(dst_hbm.at[wid], dst_v)
        pltpu.sync_copy(ew_hbm.at[wid], ew_v)
        pltpu.sync_copy(asrc_hbm, asrc_v)
        pltpu.sync_copy(adst_hbm, adst_v)

        z16 = jnp.zeros((L,), jnp.float32)

        def zrow(r, carry):
            for i in range(p // L):
                zbuf[r, pl.ds(i * L, L)] = z16
            return carry

        lax.fori_loop(0, ZR, zrow, None)

        def zcp(q, carry):
            pltpu.sync_copy(zbuf, acc.at[pl.ds(s_ax * npw + q * ZR, ZR)])
            return carry

        lax.fori_loop(0, nzc, zcp, None)
        plsc.subcore_barrier()

        c_sc = asrc_v[pl.ds(n, L)][0]

        def chunk(j, carry):
            idx_s = src_v.at[j]
            idx_d = dst_v.at[j]
            pltpu.async_copy(hpad_hbm.at[idx_s], graw, gsem).wait()
            for v in range(K // L):
                sl = pl.ds(v * L, L)
                s16 = src_v[j, sl]
                d16 = dst_v[j, sl]
                w16 = ew_v[j, sl]
                a1 = plsc.load_gather(asrc_v, [s16])
                a2 = plsc.load_gather(adst_v, [d16])
                al = a1 + a2 + c_sc * w16
                al = jnp.maximum(al, 0.2 * al)
                ex16 = jnp.exp(al)
                for l in range(L):
                    ei = v * L + l
                    x = ex16[l]
                    for i in range(nscale):
                        sli = pl.ds(i * L, L)
                        bscat[ei, sli] = graw[ei, sli] * x
            pltpu.async_copy(bscat, acc.at[idx_d], ssem, add=True).wait()
            return carry

        lax.fori_loop(0, nch, chunk, None)
        plsc.subcore_barrier()
        pltpu.sync_copy(acc.at[pl.ds(s_ax * npw, npw)], out_hbm.at[c_ax, s_ax])

    def run(src_r, dst_r, ew_r, asrc_aug, adst, hpad):
        parts = sc_kernel(src_r, dst_r, ew_r, asrc_aug, adst, hpad)
        return parts.reshape(NC, n, p)

    return run


def _make_sc_deg_pass(n, e):
    p = L
    epw = e // NW
    nch = epw // K
    npw = n // NS
    ZR = 125
    nzc = npw // ZR
    mesh = plsc.VectorSubcoreMesh(core_axis_name="c", subcore_axis_name="s")

    @functools.partial(
        pl.kernel,
        out_type=jax.ShapeDtypeStruct((NC, NS, n // NS, p), jnp.float32),
        mesh=mesh,
        compiler_params=pltpu.CompilerParams(needs_layout_passes=False,
                                             use_tc_tiling_on_sc=False),
        scratch_types=[
            pltpu.VMEM((nch, K), jnp.int32),
            pltpu.VMEM((nch, K), jnp.float32),
            pltpu.VMEM((K, p), jnp.float32),
            pltpu.VMEM((ZR, p), jnp.float32),
            pltpu.VMEM_SHARED((n, p), jnp.float32),
            pltpu.SemaphoreType.DMA,
        ],
    )
    def sc_kernel(dst_hbm, ew_hbm, out_hbm, dst_v, ew_v, bscat, zbuf, acc,
                  ssem):
        c_ax = lax.axis_index("c")
        s_ax = lax.axis_index("s")
        wid = c_ax * NS + s_ax
        pltpu.sync_copy(dst_hbm.at[wid], dst_v)
        pltpu.sync_copy(ew_hbm.at[wid], ew_v)

        z16 = jnp.zeros((L,), jnp.float32)

        def zrow(r, carry):
            zbuf[r, pl.ds(0, L)] = z16
            return carry

        lax.fori_loop(0, ZR, zrow, None)

        def zcp(q, carry):
            pltpu.sync_copy(zbuf, acc.at[pl.ds(s_ax * npw + q * ZR, ZR)])
            return carry

        lax.fori_loop(0, nzc, zcp, None)
        plsc.subcore_barrier()

        lane = lax.iota(jnp.int32, L)
        m0 = lane == 0
        m1 = lane == 1

        def chunk(j, carry):
            idx_d = dst_v.at[j]
            for v in range(K // L):
                w16 = ew_v[j, pl.ds(v * L, L)]
                for l in range(L):
                    ei = v * L + l
                    fold = jnp.where(m0, 1.0, jnp.where(m1, w16[l], 0.0))
                    bscat[ei, pl.ds(0, L)] = fold
            pltpu.async_copy(bscat, acc.at[idx_d], ssem, add=True).wait()
            return carry

        lax.fori_loop(0, nch, chunk, None)
        plsc.subcore_barrier()
        pltpu.sync_copy(acc.at[pl.ds(s_ax * npw, npw)], out_hbm.at[c_ax, s_ax])

    def run(dst_r, ew_r):
        parts = sc_kernel(dst_r, ew_r)
        return parts.reshape(NC, n, p)

    return run


def _softmax(x, axis):
    m = jnp.max(x, axis=axis, keepdims=True)
    ex = jnp.exp(x - m)
    return ex / jnp.sum(ex, axis=axis, keepdims=True)


def _gru_body(x_ref, wat_ref, wih_ref, whh_ref, h_ref, *, t_steps, f_in):
    x = x_ref[...]
    wat = wat_ref[...]
    wih = wih_ref[...]
    whh = whh_ref[...]
    h = jnp.zeros((x.shape[0], H), jnp.float32)
    for t in range(t_steps):
        attn = _softmax(jnp.dot(h, wat), axis=0)
        xt = x[:, t * f_in:(t + 1) * f_in]
        gates = jnp.dot(xt * attn, wih) + jnp.dot(h, whh)
        r = jax.nn.sigmoid(gates[:, :H])
        u = jax.nn.sigmoid(gates[:, H:])
        h = u * h + (1.0 - u) * jnp.tanh(r * h)
    h_ref[...] = h


def _proj(h, att):
    return jnp.dot(h, att[0][:, None])


def _conv1_pre_body(xg_ref, w_ref, asr_ref, ads_ref, we_ref, ae_ref,
                    hpad_ref, asrc_ref, adst_ref, cvec_ref):
    xg = xg_ref[...]
    h = jnp.dot(xg, w_ref[...])
    n = h.shape[0]
    c = jnp.sum(we_ref[...][0] * ae_ref[...][0])
    hpad_ref[...] = jnp.concatenate(
        [h, jnp.ones((n, 1), jnp.float32), jnp.zeros((n, 15), jnp.float32)],
        axis=1)
    asrc_ref[...] = _proj(h, asr_ref[...])
    adst_ref[...] = _proj(h, ads_ref[...])
    cvec_ref[...] = jnp.full((L,), c, jnp.float32)


def _conv2_pre_body(parts_ref, deg_ref, hpad1_ref, asrc1_ref,
                    asrc1_blk_ref, adst1_ref,
                    b1_ref, w2_ref, asr2_ref, ads2_ref, we2_ref, ae2_ref,
                    hpad2a_ref, hpad2b_ref, asrc2_ref, adst2_ref, exl2_ref,
                    cnt_ref, sumw_ref, cvec2_ref, *, n_total):
    ps = parts_ref[...]
    tot = ps[0] + ps[1]
    dg = deg_ref[...]
    n = tot.shape[0]
    h1g = hpad1_ref[...][:, :64]
    asrc1f = asrc1_ref[...]
    c1 = asrc1f[n_total]
    asrc1 = asrc1_blk_ref[...][:, 0]
    adst1 = adst1_ref[...][:, 0]
    cnt = dg[0, :, 0] + dg[1, :, 0]
    sumw = dg[0, :, 1] + dg[1, :, 1]
    loop_attr = sumw / jnp.maximum(cnt, 1.0)
    al = asrc1 + adst1 + c1 * loop_attr
    al = jnp.maximum(al, 0.2 * al)
    exl1 = jnp.exp(al)
    num = tot[:, :64] + exl1[:, None] * h1g
    den = tot[:, 64] + exl1
    g1 = jnp.maximum(num / (den[:, None] + 1e-16) + b1_ref[...][None, :], 0.0)
    h2 = jnp.dot(g1, w2_ref[...])
    c2 = jnp.sum(we2_ref[...][0] * ae2_ref[...][0])
    ones = jnp.ones((n, 1), jnp.float32)
    zeros = jnp.zeros((n, 15), jnp.float32)
    hpad2a_ref[...] = jnp.concatenate([h2[:, :64], ones, zeros], axis=1)
    hpad2b_ref[...] = jnp.concatenate([h2[:, 64:], ones, zeros], axis=1)
    asrc2 = _proj(h2, asr2_ref[...])
    adst2 = _proj(h2, ads2_ref[...])
    asrc2_ref[...] = asrc2
    adst2_ref[...] = adst2
    al2 = asrc2[:, 0] + adst2[:, 0] + c2 * loop_attr
    al2 = jnp.maximum(al2, 0.2 * al2)
    exl2_ref[...] = jnp.exp(al2)[:, None]
    cnt_ref[...] = cnt[:, None]
    sumw_ref[...] = sumw[:, None]
    cvec2_ref[...] = jnp.full((L,), c2, jnp.float32)


def _mid_body(partsa_ref, partsb_ref, hpad2a_ref, hpad2b_ref, exl2_ref,
              b2_ref, h1_ref,
              mhs1_ref, mhs2_ref, wq_ref, wk_ref, wv_ref, bq_ref, bk_ref,
              bv_ref, wvsa_ref, bvsa_ref, wf1_ref, asr3_ref, ads3_ref,
              we3_ref, ae3_ref, cnt_ref, sumw_ref,
              hpad3_ref, asrc3_ref, adst3_ref, exl3_ref, cvec3_ref):
    psa = partsa_ref[...]
    psb = partsb_ref[...]
    tota = psa[0] + psa[1]
    totb = psb[0] + psb[1]
    n = tota.shape[0]
    h2 = jnp.concatenate([hpad2a_ref[...][:, :64], hpad2b_ref[...][:, :64]],
                         axis=1)
    exl2 = exl2_ref[...][:, 0]
    num = jnp.concatenate([tota[:, :64], totb[:, :64]], axis=1) \
        + exl2[:, None] * h2
    den = tota[:, 64] + exl2
    g = num / (den[:, None] + 1e-16) + b2_ref[...][None, :]

    h1 = h1_ref[...]
    wq = wq_ref[...]
    wk = wk_ref[...]
    wv = wv_ref[...]
    bq = bq_ref[...]
    bk = bk_ref[...]
    bv = bv_ref[...]
    sc = H ** -0.5

    def crossrep(q_in, kv):
        q = jnp.dot(q_in, wq) + bq[None, :]
        k = jnp.dot(kv, wk) + bk[None, :]
        v = jnp.dot(kv, wv) + bv[None, :]
        aw = _softmax(jnp.dot(q, k.T) * sc, axis=1)
        return jnp.dot(aw, v)

    rep1 = crossrep(h1, mhs1_ref[...])
    rep2 = crossrep(g, mhs2_ref[...])

    wvsa = wvsa_ref[...]
    wf1 = wf1_ref[...]
    wc0 = jnp.dot(wvsa[0 * H:1 * H], wf1)
    wc1 = jnp.dot(wvsa[1 * H:2 * H], wf1)
    wc2 = jnp.dot(wvsa[2 * H:3 * H], wf1)
    wc3 = jnp.dot(wvsa[3 * H:4 * H], wf1)
    bc = jnp.dot(bvsa_ref[...][None, :], wf1)[0]
    hf1 = (jnp.dot(h1, wc0) + jnp.dot(g, wc1) + jnp.dot(rep1, wc2)
           + jnp.dot(rep2, wc3) + bc[None, :])

    c3 = jnp.sum(we3_ref[...][0] * ae3_ref[...][0])
    hpad3_ref[...] = jnp.concatenate(
        [hf1, jnp.ones((n, 1), jnp.float32), jnp.zeros((n, 15), jnp.float32)],
        axis=1)
    asrc3 = _proj(hf1, asr3_ref[...])
    adst3 = _proj(hf1, ads3_ref[...])
    asrc3_ref[...] = asrc3
    adst3_ref[...] = adst3
    cvec3_ref[...] = jnp.full((L,), c3, jnp.float32)
    loop_attr = sumw_ref[...][:, 0] / jnp.maximum(cnt_ref[...][:, 0], 1.0)
    al3 = asrc3[:, 0] + adst3[:, 0] + c3 * loop_attr
    al3 = jnp.maximum(al3, 0.2 * al3)
    exl3_ref[...] = jnp.exp(al3)[:, None]


def _conv4_pre_body(parts_ref, hpad3_ref, exl3_ref, b3_ref, w4_ref,
                    asr4_ref, ads4_ref, we4_ref, ae4_ref, cnt_ref, sumw_ref,
                    hpad4_ref, asrc4_ref, adst4_ref, exl4_ref, cvec4_ref):
    ps = parts_ref[...]
    tot = ps[0] + ps[1]
    n = tot.shape[0]
    h3 = hpad3_ref[...][:, :64]
    exl3 = exl3_ref[...][:, 0]
    num = tot[:, :64] + exl3[:, None] * h3
    den = tot[:, 64] + exl3
    y1 = jnp.maximum(num / (den[:, None] + 1e-16) + b3_ref[...][None, :], 0.0)
    h4 = jnp.dot(y1, w4_ref[...])
    c4 = jnp.sum(we4_ref[...][0] * ae4_ref[...][0])
    hpad4_ref[...] = jnp.concatenate(
        [h4, jnp.ones((n, 1), jnp.float32), jnp.zeros((n, 14), jnp.float32)],
        axis=1)
    asrc4 = _proj(h4, asr4_ref[...])
    adst4 = _proj(h4, ads4_ref[...])
    asrc4_ref[...] = asrc4
    adst4_ref[...] = adst4
    cvec4_ref[...] = jnp.full((L,), c4, jnp.float32)
    loop_attr = sumw_ref[...][:, 0] / jnp.maximum(cnt_ref[...][:, 0], 1.0)
    al4 = asrc4[:, 0] + adst4[:, 0] + c4 * loop_attr
    al4 = jnp.maximum(al4, 0.2 * al4)
    exl4_ref[...] = jnp.exp(al4)[:, None]


def _final_body(parts_ref, hpad4_ref, exl4_ref, b4_ref, y_ref):
    ps = parts_ref[...]
    tot = ps[0] + ps[1]
    h4 = hpad4_ref[...][:, 0]
    exl4 = exl4_ref[...][:, 0]
    num = tot[:, 0] + exl4 * h4
    den = tot[:, 1] + exl4
    y = num / (den + 1e-16) + b4_ref[...][0]
    y_ref[...] = jnp.maximum(y, 0.0)[:, None]


def _tc_call(body, out_shapes, *args, **kw):
    return pl.pallas_call(
        functools.partial(body, **kw),
        out_shape=out_shapes,
    )(*args)


G = 10


def _tc_blocked(body, n, out_shapes, *args, **kw):
    nb = n // G

    def spec(shape):
        if n in shape:
            ax = shape.index(n)
            bshape = tuple(nb if i == ax else s for i, s in enumerate(shape))

            def im(i, ax=ax, r=len(shape)):
                return tuple(i if j == ax else 0 for j in range(r))

            return pl.BlockSpec(bshape, im)
        r = len(shape)
        return pl.BlockSpec(shape, lambda i, r=r: (0,) * r)

    in_specs = [spec(a.shape) for a in args]
    out_specs = jax.tree.map(lambda t: spec(t.shape), out_shapes)
    return pl.pallas_call(
        functools.partial(body, **kw),
        grid=(G,),
        in_specs=in_specs,
        out_specs=out_specs,
        out_shape=out_shapes,
    )(*args)


def kernel(x_time_series, x_graph, edge_index, edge_weight, params):
    n = x_graph.shape[0]
    e = edge_index.shape[1]
    t_steps = x_time_series.shape[2]
    f_in = x_time_series.shape[3]
    nch = e // (NW * K)

    src_r = edge_index[0].astype(jnp.int32).reshape(NW, nch, K)
    dst_r = edge_index[1].astype(jnp.int32).reshape(NW, nch, K)
    ew_r = edge_weight[:, 0].reshape(NW, nch, K)
    xflat = x_time_series[-1].reshape(n, t_steps * f_in)

    f32 = jnp.float32
    sd = jax.ShapeDtypeStruct

    gp = params["gru"]
    h1 = _tc_call(_gru_body, sd((n, H), f32),
                  xflat, gp["W_attn"], gp["W_ih"], gp["W_hh"],
                  t_steps=t_steps, f_in=f_in)

    degparts = _make_sc_deg_pass(n, e)(dst_r, ew_r)

    g1p = params["gat1"]
    hpad1, asrc1, adst1, cvec1 = _tc_blocked(
        _conv1_pre_body, n,
        (sd((n, 80), f32), sd((n, 1), f32), sd((n, 1), f32), sd((L,), f32)),
        x_graph, g1p["W"], g1p["att_src"], g1p["att_dst"], g1p["W_e"],
        g1p["att_edge"])
    asrc1_aug = jnp.concatenate([asrc1[:, 0], cvec1])

    sc1 = _make_sc_edge_pass(n, e, 80, 64)
    parts1 = sc1(src_r, dst_r, ew_r, asrc1_aug, adst1[:, 0], hpad1)

    g2p = params["gat2"]
    (hpad2a, hpad2b, asrc2, adst2, exl2, cnt, sumw, cvec2) = _tc_blocked(
        _conv2_pre_body, n,
        (sd((n, 80), f32), sd((n, 80), f32), sd((n, 1), f32),
         sd((n, 1), f32), sd((n, 1), f32), sd((n, 1), f32), sd((n, 1), f32),
         sd((L,), f32)),
        parts1, degparts, hpad1, asrc1_aug, asrc1, adst1, g1p["bias"],
        g2p["W"], g2p["att_src"], g2p["att_dst"], g2p["W_e"], g2p["att_edge"],
        n_total=n)
    asrc2_aug = jnp.concatenate([asrc2[:, 0], cvec2])

    sc2 = _make_sc_edge_pass(n, e, 80, 64)
    parts2a = sc2(src_r, dst_r, ew_r, asrc2_aug, adst2[:, 0], hpad2a)
    parts2b = sc2(src_r, dst_r, ew_r, asrc2_aug, adst2[:, 0], hpad2b)

    ca = params["ca"]
    f1p = params["fgat1"]
    (hpad3, asrc3, adst3, exl3, cvec3) = _tc_blocked(
        _mid_body, n,
        (sd((n, 80), f32), sd((n, 1), f32), sd((n, 1), f32), sd((n, 1), f32),
         sd((L,), f32)),
        parts2a, parts2b, hpad2a, hpad2b, exl2, g2p["bias"], h1,
        params["mhs1"], params["mhs2"], ca["Wq"], ca["Wk"], ca["Wv"],
        ca["bq"], ca["bk"], ca["bv"], params["sa"]["Wv"], params["sa"]["bv"],
        f1p["W"], f1p["att_src"], f1p["att_dst"], f1p["W_e"], f1p["att_edge"],
        cnt, sumw)
    asrc3_aug = jnp.concatenate([asrc3[:, 0], cvec3])

    sc3 = sc2
    parts3 = sc3(src_r, dst_r, ew_r, asrc3_aug, adst3[:, 0], hpad3)

    f2p = params["fgat2"]
    (hpad4, asrc4, adst4, exl4, cvec4) = _tc_blocked(
        _conv4_pre_body, n,
        (sd((n, L), f32), sd((n, 1), f32), sd((n, 1), f32), sd((n, 1), f32),
         sd((L,), f32)),
        parts3, hpad3, exl3, f1p["bias"],
        f2p["W"], f2p["att_src"], f2p["att_dst"], f2p["W_e"], f2p["att_edge"],
        cnt, sumw)
    asrc4_aug = jnp.concatenate([asrc4[:, 0], cvec4])

    sc4 = _make_sc_edge_pass(n, e, L, 1)
    parts4 = sc4(src_r, dst_r, ew_r, asrc4_aug, adst4[:, 0], hpad4)

    y = _tc_blocked(_final_body, n, sd((n, 1), f32),
                    parts4, hpad4, exl4, f2p["bias"])
    return y[:, 0]


# speedup vs baseline: 48.9643x; 1.3032x over previous
import functools

import jax
import jax.numpy as jnp
from jax import lax
from jax.experimental import pallas as pl
from jax.experimental.pallas import tpu as pltpu
from jax.experimental.pallas import tpu_sc as plsc

H = 128
NC, NS, L = 2, 16, 16
NW = NC * NS
K = 80


def _make_sc_wide_pass(n, e, with_deg):
    dg = 64
    p = dg + L
    epw = e // NW
    nch = epw // K
    npw = n // NS
    ZR = 125
    nzc = npw // ZR
    mesh = plsc.VectorSubcoreMesh(core_axis_name="c", subcore_axis_name="s")

    @functools.partial(
        pl.kernel,
        out_type=jax.ShapeDtypeStruct((NC, NS, n // NS, p), jnp.float32),
        mesh=mesh,
        compiler_params=pltpu.CompilerParams(needs_layout_passes=False,
                                             use_tc_tiling_on_sc=False),
        scratch_types=[
            pltpu.VMEM((nch, K), jnp.int32),
            pltpu.VMEM((nch, K), jnp.int32),
            pltpu.VMEM((nch, K), jnp.float32),
            pltpu.VMEM((n + L,), jnp.float32),
            pltpu.VMEM((n,), jnp.float32),
            pltpu.VMEM((K, dg), jnp.float32),
            pltpu.VMEM((K, p), jnp.float32),
            pltpu.VMEM((ZR, p), jnp.float32),
            pltpu.VMEM_SHARED((n, p), jnp.float32),
            pltpu.SemaphoreType.DMA,
            pltpu.SemaphoreType.DMA,
        ],
    )
    def sc_kernel(src_hbm, dst_hbm, ew_hbm, asrc_hbm, adst_hbm, h_hbm,
                  out_hbm, src_v, dst_v, ew_v, asrc_v, adst_v, graw,
                  bscat, zbuf, acc, gsem, ssem):
        c_ax = lax.axis_index("c")
        s_ax = lax.axis_index("s")
        wid = c_ax * NS + s_ax
        pltpu.sync_copy(src_hbm.at[wid], src_v)
        pltpu.sync_copy(dst_hbm.at[wid], dst_v)
        pltpu.sync_copy(ew_hbm.at[wid], ew_v)
        pltpu.sync_copy(asrc_hbm, asrc_v)
        pltpu.sync_copy(adst_hbm, adst_v)

        z16 = jnp.zeros((L,), jnp.float32)

        def zrow(r, carry):
            for i in range(p // L):
                zbuf[r, pl.ds(i * L, L)] = z16
            return carry

        lax.fori_loop(0, ZR, zrow, None)

        def zcp(q, carry):
            pltpu.sync_copy(zbuf, acc.at[pl.ds(s_ax * npw + q * ZR, ZR)])
            return carry

        lax.fori_loop(0, nzc, zcp, None)
        plsc.subcore_barrier()

        c_sc = asrc_v[pl.ds(n, L)][0]
        lane = lax.iota(jnp.int32, L)
        m0 = lane == 0
        m1 = lane == 1
        m2 = lane == 2

        def chunk(j, carry):
            idx_s = src_v.at[j]
            idx_d = dst_v.at[j]
            pltpu.async_copy(h_hbm.at[idx_s], graw, gsem).wait()
            for v in range(K // L):
                sl = pl.ds(v * L, L)
                s16 = src_v[j, sl]
                d16 = dst_v[j, sl]
                w16 = ew_v[j, sl]
                a1 = plsc.load_gather(asrc_v, [s16])
                a2 = plsc.load_gather(adst_v, [d16])
                al = a1 + a2 + c_sc * w16
                al = jnp.maximum(al, 0.2 * al)
                ex16 = jnp.exp(al)
                for l in range(L):
                    ei = v * L + l
                    x = ex16[l]
                    for i in range(dg // L):
                        sli = pl.ds(i * L, L)
                        bscat[ei, sli] = graw[ei, sli] * x
                    if with_deg:
                        aux = jnp.where(
                            m0, x, jnp.where(
                                m1, 1.0, jnp.where(m2, w16[l], 0.0)))
                    else:
                        aux = jnp.where(m0, x, 0.0)
                    bscat[ei, pl.ds(dg, L)] = aux
            pltpu.async_copy(bscat, acc.at[idx_d], ssem, add=True).wait()
            return carry

        lax.fori_loop(0, nch, chunk, None)
        plsc.subcore_barrier()
        pltpu.sync_copy(acc.at[pl.ds(s_ax * npw, npw)], out_hbm.at[c_ax, s_ax])

    def run(src_r, dst_r, ew_r, asrc_aug, adst, h):
        parts = sc_kernel(src_r, dst_r, ew_r, asrc_aug, adst, h)
        return parts.reshape(NC, n, p)

    return run


def _make_sc_narrow_pass(n, e):
    p = L
    d = 1
    epw = e // NW
    nch = epw // K
    npw = n // NS
    ZR = 125
    nzc = npw // ZR
    nscale = (d + 1 + (L - 1)) // L
    mesh = plsc.VectorSubcoreMesh(core_axis_name="c", subcore_axis_name="s")

    @functools.partial(
        pl.kernel,
        out_type=jax.ShapeDtypeStruct((NC, NS, n // NS, p), jnp.float32),
        mesh=mesh,
        compiler_params=pltpu.CompilerParams(needs_layout_passes=False,
                                             use_tc_tiling_on_sc=False),
        scratch_types=[
            pltpu.VMEM((nch, K), jnp.int32),
            pltpu.VMEM((nch, K), jnp.int32),
            pltpu.VMEM((nch, K), jnp.float32),
            pltpu.VMEM((n + L,), jnp.float32),
            pltpu.VMEM((n,), jnp.float32),
            pltpu.VMEM((K, p), jnp.float32),
            pltpu.VMEM((K, p), jnp.float32),
            pltpu.VMEM((ZR, p), jnp.float32),
            pltpu.VMEM_SHARED((n, p), jnp.float32),
            pltpu.SemaphoreType.DMA,
            pltpu.SemaphoreType.DMA,
        ],
    )
    def sc_kernel(src_hbm, dst_hbm, ew_hbm, asrc_hbm, adst_hbm, hpad_hbm,
                  out_hbm, src_v, dst_v, ew_v, asrc_v, adst_v, graw,
                  bscat, zbuf, acc, gsem, ssem):
        c_ax = lax.axis_index("c")
        s_ax = lax.axis_index("s")
        wid = c_ax * NS + s_ax
        pltpu.sync_copy(src_hbm.at[wid], src_v)
        pltpu.sync_copy(dst_hbm.at[wid], dst_v)
        pltpu.sync_copy(ew_hbm.at[wid], ew_v)
        pltpu.sync_copy(asrc_hbm, asrc_v)
        pltpu.sync_copy(adst_hbm, adst_v)

        z16 = jnp.zeros((L,), jnp.float32)

        def zrow(r, carry):
            zbuf[r, pl.ds(0, L)] = z16
            return carry

        lax.fori_loop(0, ZR, zrow, None)

        def zcp(q, carry):
            pltpu.sync_copy(zbuf, acc.at[pl.ds(s_ax * npw + q * ZR, ZR)])
            return carry

        lax.fori_loop(0, nzc, zcp, None)
        plsc.subcore_barrier()

        c_sc = asrc_v[pl.ds(n, L)][0]

        def chunk(j, carry):
            idx_s = src_v.at[j]
            idx_d = dst_v.at[j]
            pltpu.async_copy(hpad_hbm.at[idx_s], graw, gsem).wait()
            for v in range(K // L):
                sl = pl.ds(v * L, L)
                s16 = src_v[j, sl]
                d16 = dst_v[j, sl]
                w16 = ew_v[j, sl]
                a1 = plsc.load_gather(asrc_v, [s16])
                a2 = plsc.load_gather(adst_v, [d16])
                al = a1 + a2 + c_sc * w16
                al = jnp.maximum(al, 0.2 * al)
                ex16 = jnp.exp(al)
                for l in range(L):
                    ei = v * L + l
                    x = ex16[l]
                    for i in range(nscale):
                        sli = pl.ds(i * L, L)
                        bscat[ei, sli] = graw[ei, sli] * x
            pltpu.async_copy(bscat, acc.at[idx_d], ssem, add=True).wait()
            return carry

        lax.fori_loop(0, nch, chunk, None)
        plsc.subcore_barrier()
        pltpu.sync_copy(acc.at[pl.ds(s_ax * npw, npw)], out_hbm.at[c_ax, s_ax])

    def run(src_r, dst_r, ew_r, asrc_aug, adst, hpad):
        parts = sc_kernel(src_r, dst_r, ew_r, asrc_aug, adst, hpad)
        return parts.reshape(NC, n, p)

    return run


def _softmax(x, axis):
    m = jnp.max(x, axis=axis, keepdims=True)
    ex = jnp.exp(x - m)
    return ex / jnp.sum(ex, axis=axis, keepdims=True)


def _gru_body(x_ref, wat_ref, wih_ref, whh_ref, h_ref, *, t_steps, f_in):
    x = x_ref[...]
    wat = wat_ref[...]
    wih = wih_ref[...]
    whh = whh_ref[...]
    h = jnp.zeros((x.shape[0], H), jnp.float32)
    for t in range(t_steps):
        attn = _softmax(jnp.dot(h, wat), axis=0)
        xt = x[:, t * f_in:(t + 1) * f_in]
        gates = jnp.dot(xt * attn, wih) + jnp.dot(h, whh)
        r = jax.nn.sigmoid(gates[:, :H])
        u = jax.nn.sigmoid(gates[:, H:])
        h = u * h + (1.0 - u) * jnp.tanh(r * h)
    h_ref[...] = h


def _proj(h, att):
    return jnp.dot(h, att[0][:, None])


def _conv1_pre_body(xg_ref, w_ref, asr_ref, ads_ref, we_ref, ae_ref,
                    h_out_ref, asrc_ref, adst_ref, cvec_ref):
    xg = xg_ref[...]
    h = jnp.dot(xg, w_ref[...])
    c = jnp.sum(we_ref[...][0] * ae_ref[...][0])
    h_out_ref[...] = h
    asrc_ref[...] = _proj(h, asr_ref[...])
    adst_ref[...] = _proj(h, ads_ref[...])
    cvec_ref[...] = jnp.full((L,), c, jnp.float32)


def _conv2_pre_body(parts_ref, h1g_ref, asrc1_ref, asrc1_blk_ref, adst1_ref,
                    b1_ref, w2_ref, asr2_ref, ads2_ref, we2_ref, ae2_ref,
                    g1_ref, asrc2_ref, adst2_ref, exl2_ref,
                    cnt_ref, sumw_ref, cvec2_ref, *, n_total):
    ps = parts_ref[...]
    tot = ps[0] + ps[1]
    h1g = h1g_ref[...]
    asrc1f = asrc1_ref[...]
    c1 = asrc1f[n_total]
    asrc1 = asrc1_blk_ref[...][:, 0]
    adst1 = adst1_ref[...][:, 0]
    cnt = tot[:, 65]
    sumw = tot[:, 66]
    loop_attr = sumw / jnp.maximum(cnt, 1.0)
    al = asrc1 + adst1 + c1 * loop_attr
    al = jnp.maximum(al, 0.2 * al)
    exl1 = jnp.exp(al)
    num = tot[:, :64] + exl1[:, None] * h1g
    den = tot[:, 64] + exl1
    g1 = jnp.maximum(num / (den[:, None] + 1e-16) + b1_ref[...][None, :], 0.0)
    h2 = jnp.dot(g1, w2_ref[...])
    c2 = jnp.sum(we2_ref[...][0] * ae2_ref[...][0])
    asrc2 = _proj(h2, asr2_ref[...])
    adst2 = _proj(h2, ads2_ref[...])
    g1_ref[...] = g1
    asrc2_ref[...] = asrc2
    adst2_ref[...] = adst2
    al2 = asrc2[:, 0] + adst2[:, 0] + c2 * loop_attr
    al2 = jnp.maximum(al2, 0.2 * al2)
    exl2_ref[...] = jnp.exp(al2)[:, None]
    cnt_ref[...] = cnt[:, None]
    sumw_ref[...] = sumw[:, None]
    cvec2_ref[...] = jnp.full((L,), c2, jnp.float32)


def _mid_body(parts_ref, g1_ref, exl2_ref, w2_ref, b2_ref, h1_ref,
              mhs1_ref, mhs2_ref, wq_ref, wk_ref, wv_ref, bq_ref, bk_ref,
              bv_ref, wvsa_ref, bvsa_ref, wf1_ref, asr3_ref, ads3_ref,
              we3_ref, ae3_ref, cnt_ref, sumw_ref,
              hf1_ref, asrc3_ref, adst3_ref, exl3_ref, cvec3_ref):
    ps = parts_ref[...]
    tot = ps[0] + ps[1]
    g1 = g1_ref[...]
    exl2 = exl2_ref[...][:, 0]
    num = tot[:, :64] + exl2[:, None] * g1
    den = tot[:, 64] + exl2
    aggpre = num / (den[:, None] + 1e-16)
    g = jnp.dot(aggpre, w2_ref[...]) + b2_ref[...][None, :]

    h1 = h1_ref[...]
    wq = wq_ref[...]
    wk = wk_ref[...]
    wv = wv_ref[...]
    bq = bq_ref[...]
    bk = bk_ref[...]
    bv = bv_ref[...]
    sc = H ** -0.5

    def crossrep(q_in, kv):
        q = jnp.dot(q_in, wq) + bq[None, :]
        k = jnp.dot(kv, wk) + bk[None, :]
        v = jnp.dot(kv, wv) + bv[None, :]
        aw = _softmax(jnp.dot(q, k.T) * sc, axis=1)
        return jnp.dot(aw, v)

    rep1 = crossrep(h1, mhs1_ref[...])
    rep2 = crossrep(g, mhs2_ref[...])

    wvsa = wvsa_ref[...]
    wf1 = wf1_ref[...]
    wc0 = jnp.dot(wvsa[0 * H:1 * H], wf1)
    wc1 = jnp.dot(wvsa[1 * H:2 * H], wf1)
    wc2 = jnp.dot(wvsa[2 * H:3 * H], wf1)
    wc3 = jnp.dot(wvsa[3 * H:4 * H], wf1)
    bc = jnp.dot(bvsa_ref[...][None, :], wf1)[0]
    hf1 = (jnp.dot(h1, wc0) + jnp.dot(g, wc1) + jnp.dot(rep1, wc2)
           + jnp.dot(rep2, wc3) + bc[None, :])

    c3 = jnp.sum(we3_ref[...][0] * ae3_ref[...][0])
    hf1_ref[...] = hf1
    asrc3 = _proj(hf1, asr3_ref[...])
    adst3 = _proj(hf1, ads3_ref[...])
    asrc3_ref[...] = asrc3
    adst3_ref[...] = adst3
    cvec3_ref[...] = jnp.full((L,), c3, jnp.float32)
    loop_attr = sumw_ref[...][:, 0] / jnp.maximum(cnt_ref[...][:, 0], 1.0)
    al3 = asrc3[:, 0] + adst3[:, 0] + c3 * loop_attr
    al3 = jnp.maximum(al3, 0.2 * al3)
    exl3_ref[...] = jnp.exp(al3)[:, None]


def _conv4_pre_body(parts_ref, hf1_ref, exl3_ref, b3_ref, w4_ref,
                    asr4_ref, ads4_ref, we4_ref, ae4_ref, cnt_ref, sumw_ref,
                    hpad4_ref, asrc4_ref, adst4_ref, exl4_ref, cvec4_ref):
    ps = parts_ref[...]
    tot = ps[0] + ps[1]
    n = tot.shape[0]
    h3 = hf1_ref[...]
    exl3 = exl3_ref[...][:, 0]
    num = tot[:, :64] + exl3[:, None] * h3
    den = tot[:, 64] + exl3
    y1 = jnp.maximum(num / (den[:, None] + 1e-16) + b3_ref[...][None, :], 0.0)
    h4 = jnp.dot(y1, w4_ref[...])
    c4 = jnp.sum(we4_ref[...][0] * ae4_ref[...][0])
    hpad4_ref[...] = jnp.concatenate(
        [h4, jnp.ones((n, 1), jnp.float32), jnp.zeros((n, 14), jnp.float32)],
        axis=1)
    asrc4 = _proj(h4, asr4_ref[...])
    adst4 = _proj(h4, ads4_ref[...])
    asrc4_ref[...] = asrc4
    adst4_ref[...] = adst4
    cvec4_ref[...] = jnp.full((L,), c4, jnp.float32)
    loop_attr = sumw_ref[...][:, 0] / jnp.maximum(cnt_ref[...][:, 0], 1.0)
    al4 = asrc4[:, 0] + adst4[:, 0] + c4 * loop_attr
    al4 = jnp.maximum(al4, 0.2 * al4)
    exl4_ref[...] = jnp.exp(al4)[:, None]


def _final_body(parts_ref, hpad4_ref, exl4_ref, b4_ref, y_ref):
    ps = parts_ref[...]
    tot = ps[0] + ps[1]
    h4 = hpad4_ref[...][:, 0]
    exl4 = exl4_ref[...][:, 0]
    num = tot[:, 0] + exl4 * h4
    den = tot[:, 1] + exl4
    y = num / (den + 1e-16) + b4_ref[...][0]
    y_ref[...] = jnp.maximum(y, 0.0)[:, None]


def _tc_call(body, out_shapes, *args, **kw):
    return pl.pallas_call(
        functools.partial(body, **kw),
        out_shape=out_shapes,
    )(*args)


G = 10


def _tc_blocked(body, n, out_shapes, *args, **kw):
    nb = n // G

    def spec(shape):
        if n in shape:
            ax = shape.index(n)
            bshape = tuple(nb if i == ax else s for i, s in enumerate(shape))

            def im(i, ax=ax, r=len(shape)):
                return tuple(i if j == ax else 0 for j in range(r))

            return pl.BlockSpec(bshape, im)
        r = len(shape)
        return pl.BlockSpec(shape, lambda i, r=r: (0,) * r)

    in_specs = [spec(a.shape) for a in args]
    out_specs = jax.tree.map(lambda t: spec(t.shape), out_shapes)
    return pl.pallas_call(
        functools.partial(body, **kw),
        grid=(G,),
        in_specs=in_specs,
        out_specs=out_specs,
        out_shape=out_shapes,
    )(*args)


def kernel(x_time_series, x_graph, edge_index, edge_weight, params):
    n = x_graph.shape[0]
    e = edge_index.shape[1]
    t_steps = x_time_series.shape[2]
    f_in = x_time_series.shape[3]
    nch = e // (NW * K)

    src_r = edge_index[0].astype(jnp.int32).reshape(NW, nch, K)
    dst_r = edge_index[1].astype(jnp.int32).reshape(NW, nch, K)
    ew_r = edge_weight[:, 0].reshape(NW, nch, K)
    xflat = x_time_series[-1].reshape(n, t_steps * f_in)

    f32 = jnp.float32
    sd = jax.ShapeDtypeStruct

    gp = params["gru"]
    h1 = _tc_call(_gru_body, sd((n, H), f32),
                  xflat, gp["W_attn"], gp["W_ih"], gp["W_hh"],
                  t_steps=t_steps, f_in=f_in)

    g1p = params["gat1"]
    h1g, asrc1, adst1, cvec1 = _tc_blocked(
        _conv1_pre_body, n,
        (sd((n, 64), f32), sd((n, 1), f32), sd((n, 1), f32), sd((L,), f32)),
        x_graph, g1p["W"], g1p["att_src"], g1p["att_dst"], g1p["W_e"],
        g1p["att_edge"])
    asrc1_aug = jnp.concatenate([asrc1[:, 0], cvec1])

    sc_deg = _make_sc_wide_pass(n, e, with_deg=True)
    parts1 = sc_deg(src_r, dst_r, ew_r, asrc1_aug, adst1[:, 0], h1g)

    g2p = params["gat2"]
    (g1, asrc2, adst2, exl2, cnt, sumw, cvec2) = _tc_blocked(
        _conv2_pre_body, n,
        (sd((n, 64), f32), sd((n, 1), f32), sd((n, 1), f32), sd((n, 1), f32),
         sd((n, 1), f32), sd((n, 1), f32), sd((L,), f32)),
        parts1, h1g, asrc1_aug, asrc1, adst1, g1p["bias"],
        g2p["W"], g2p["att_src"], g2p["att_dst"], g2p["W_e"], g2p["att_edge"],
        n_total=n)
    asrc2_aug = jnp.concatenate([asrc2[:, 0], cvec2])

    sc_wide = _make_sc_wide_pass(n, e, with_deg=False)
    parts2 = sc_wide(src_r, dst_r, ew_r, asrc2_aug, adst2[:, 0], g1)

    ca = params["ca"]
    f1p = params["fgat1"]
    (hf1, asrc3, adst3, exl3, cvec3) = _tc_blocked(
        _mid_body, n,
        (sd((n, 64), f32), sd((n, 1), f32), sd((n, 1), f32), sd((n, 1), f32),
         sd((L,), f32)),
        parts2, g1, exl2, g2p["W"], g2p["bias"], h1,
        params["mhs1"], params["mhs2"], ca["Wq"], ca["Wk"], ca["Wv"],
        ca["bq"], ca["bk"], ca["bv"], params["sa"]["Wv"], params["sa"]["bv"],
        f1p["W"], f1p["att_src"], f1p["att_dst"], f1p["W_e"], f1p["att_edge"],
        cnt, sumw)
    asrc3_aug = jnp.concatenate([asrc3[:, 0], cvec3])

    parts3 = sc_wide(src_r, dst_r, ew_r, asrc3_aug, adst3[:, 0], hf1)

    f2p = params["fgat2"]
    (hpad4, asrc4, adst4, exl4, cvec4) = _tc_blocked(
        _conv4_pre_body, n,
        (sd((n, L), f32), sd((n, 1), f32), sd((n, 1), f32), sd((n, 1), f32),
         sd((L,), f32)),
        parts3, hf1, exl3, f1p["bias"],
        f2p["W"], f2p["att_src"], f2p["att_dst"], f2p["W_e"], f2p["att_edge"],
        cnt, sumw)
    asrc4_aug = jnp.concatenate([asrc4[:, 0], cvec4])

    sc4 = _make_sc_narrow_pass(n, e)
    parts4 = sc4(src_r, dst_r, ew_r, asrc4_aug, adst4[:, 0], hpad4)

    y = _tc_blocked(_final_body, n, sd((n, 1), f32),
                    parts4, hpad4, exl4, f2p["bias"])
    return y[:, 0]


# speedup vs baseline: 70.1428x; 1.4325x over previous
import functools

import jax
import jax.numpy as jnp
from jax import lax
from jax.experimental import pallas as pl
from jax.experimental.pallas import tpu as pltpu
from jax.experimental.pallas import tpu_sc as plsc

H = 128
NC, NS, L = 2, 16, 16
NW = NC * NS
K = 80


def _make_sc_wide_pass(n, e, with_deg):
    dg = 64
    p = dg + L
    epw = e // NW
    nch = epw // K
    npw = n // NS
    ZR = 25
    nzc = npw // ZR
    mesh = plsc.VectorSubcoreMesh(core_axis_name="c", subcore_axis_name="s")

    @functools.partial(
        pl.kernel,
        out_type=jax.ShapeDtypeStruct((NC, NS, n // NS, p), jnp.float32),
        mesh=mesh,
        compiler_params=pltpu.CompilerParams(needs_layout_passes=False,
                                             use_tc_tiling_on_sc=False),
        scratch_types=[
            pltpu.VMEM((nch, K), jnp.int32),
            pltpu.VMEM((nch, K), jnp.int32),
            pltpu.VMEM((nch, K), jnp.float32),
            pltpu.VMEM((n + L,), jnp.float32),
            pltpu.VMEM((n,), jnp.float32),
            pltpu.VMEM((K, dg), jnp.float32),
            pltpu.VMEM((K, dg), jnp.float32),
            pltpu.VMEM((K, p), jnp.float32),
            pltpu.VMEM((ZR, p), jnp.float32),
            pltpu.VMEM_SHARED((n, p), jnp.float32),
            pltpu.SemaphoreType.DMA,
            pltpu.SemaphoreType.DMA,
            pltpu.SemaphoreType.DMA,
        ],
    )
    def sc_kernel(src_hbm, dst_hbm, ew_hbm, asrc_hbm, adst_hbm, h_hbm,
                  out_hbm, src_v, dst_v, ew_v, asrc_v, adst_v, graw0,
                  graw1, bscat, zbuf, acc, gsem0, gsem1, ssem):
        c_ax = lax.axis_index("c")
        s_ax = lax.axis_index("s")
        wid = c_ax * NS + s_ax
        pltpu.sync_copy(src_hbm.at[wid], src_v)
        pltpu.sync_copy(dst_hbm.at[wid], dst_v)
        pltpu.sync_copy(ew_hbm.at[wid], ew_v)
        pltpu.sync_copy(asrc_hbm, asrc_v)
        pltpu.sync_copy(adst_hbm, adst_v)

        z16 = jnp.zeros((L,), jnp.float32)

        def zrow(r, carry):
            for i in range(p // L):
                zbuf[r, pl.ds(i * L, L)] = z16
            return carry

        lax.fori_loop(0, ZR, zrow, None)

        def zcp(q, carry):
            pltpu.sync_copy(zbuf, acc.at[pl.ds(s_ax * npw + q * ZR, ZR)])
            return carry

        lax.fori_loop(0, nzc, zcp, None)
        plsc.subcore_barrier()

        c_sc = asrc_v[pl.ds(n, L)][0]
        lane = lax.iota(jnp.int32, L)
        m0 = lane == 0
        m1 = lane == 1
        m2 = lane == 2

        pltpu.async_copy(h_hbm.at[src_v.at[0]], graw0, gsem0)

        def chunk(j, graw, gsem):
            idx_d = dst_v.at[j]
            pltpu.make_async_copy(h_hbm.at[src_v.at[j]], graw, gsem).wait()
            for v in range(K // L):
                sl = pl.ds(v * L, L)
                s16 = src_v[j, sl]
                d16 = dst_v[j, sl]
                w16 = ew_v[j, sl]
                a1 = plsc.load_gather(asrc_v, [s16])
                a2 = plsc.load_gather(adst_v, [d16])
                al = a1 + a2 + c_sc * w16
                al = jnp.maximum(al, 0.2 * al)
                ex16 = jnp.exp(al)
                for l in range(L):
                    ei = v * L + l
                    x = ex16[l]
                    for i in range(dg // L):
                        sli = pl.ds(i * L, L)
                        bscat[ei, sli] = graw[ei, sli] * x
                    if with_deg:
                        aux = jnp.where(
                            m0, x, jnp.where(
                                m1, 1.0, jnp.where(m2, w16[l], 0.0)))
                    else:
                        aux = jnp.where(m0, x, 0.0)
                    bscat[ei, pl.ds(dg, L)] = aux
            pltpu.async_copy(bscat, acc.at[idx_d], ssem, add=True).wait()

        def ring(it, carry):
            j = it * 2
            pltpu.async_copy(h_hbm.at[src_v.at[j + 1]], graw1, gsem1)
            chunk(j, graw0, gsem0)
            pltpu.async_copy(h_hbm.at[src_v.at[j + 2]], graw0, gsem0)
            chunk(j + 1, graw1, gsem1)
            return carry

        lax.fori_loop(0, (nch - 1) // 2, ring, None)
        chunk(nch - 1, graw0, gsem0)
        plsc.subcore_barrier()
        pltpu.sync_copy(acc.at[pl.ds(s_ax * npw, npw)], out_hbm.at[c_ax, s_ax])

    def run(src_r, dst_r, ew_r, asrc_aug, adst, h):
        parts = sc_kernel(src_r, dst_r, ew_r, asrc_aug, adst, h)
        return parts.reshape(NC, n, p)

    return run


def _make_sc_narrow_pass(n, e):
    p = L
    d = 1
    epw = e // NW
    nch = epw // K
    npw = n // NS
    ZR = 25
    nzc = npw // ZR
    nscale = (d + 1 + (L - 1)) // L
    mesh = plsc.VectorSubcoreMesh(core_axis_name="c", subcore_axis_name="s")

    @functools.partial(
        pl.kernel,
        out_type=jax.ShapeDtypeStruct((NC, NS, n // NS, p), jnp.float32),
        mesh=mesh,
        compiler_params=pltpu.CompilerParams(needs_layout_passes=False,
                                             use_tc_tiling_on_sc=False),
        scratch_types=[
            pltpu.VMEM((nch, K), jnp.int32),
            pltpu.VMEM((nch, K), jnp.int32),
            pltpu.VMEM((nch, K), jnp.float32),
            pltpu.VMEM((n + L,), jnp.float32),
            pltpu.VMEM((n,), jnp.float32),
            pltpu.VMEM((K, p), jnp.float32),
            pltpu.VMEM((K, p), jnp.float32),
            pltpu.VMEM((K, p), jnp.float32),
            pltpu.VMEM((ZR, p), jnp.float32),
            pltpu.VMEM_SHARED((n, p), jnp.float32),
            pltpu.SemaphoreType.DMA,
            pltpu.SemaphoreType.DMA,
            pltpu.SemaphoreType.DMA,
        ],
    )
    def sc_kernel(src_hbm, dst_hbm, ew_hbm, asrc_hbm, adst_hbm, hpad_hbm,
                  out_hbm, src_v, dst_v, ew_v, asrc_v, adst_v, graw0,
                  graw1, bscat, zbuf, acc, gsem0, gsem1, ssem):
        c_ax = lax.axis_index("c")
        s_ax = lax.axis_index("s")
        wid = c_ax * NS + s_ax
        pltpu.sync_copy(src_hbm.at[wid], src_v)
        pltpu.sync_copy(dst_hbm.at[wid], dst_v)
        pltpu.sync_copy(ew_hbm.at[wid], ew_v)
        pltpu.sync_copy(asrc_hbm, asrc_v)
        pltpu.sync_copy(adst_hbm, adst_v)

        z16 = jnp.zeros((L,), jnp.float32)

        def zrow(r, carry):
            zbuf[r, pl.ds(0, L)] = z16
            return carry

        lax.fori_loop(0, ZR, zrow, None)

        def zcp(q, carry):
            pltpu.sync_copy(zbuf, acc.at[pl.ds(s_ax * npw + q * ZR, ZR)])
            return carry

        lax.fori_loop(0, nzc, zcp, None)
        plsc.subcore_barrier()

        c_sc = asrc_v[pl.ds(n, L)][0]

        pltpu.async_copy(hpad_hbm.at[src_v.at[0]], graw0, gsem0)

        def chunk(j, graw, gsem):
            idx_d = dst_v.at[j]
            pltpu.make_async_copy(hpad_hbm.at[src_v.at[j]], graw, gsem).wait()
            for v in range(K // L):
                sl = pl.ds(v * L, L)
                s16 = src_v[j, sl]
                d16 = dst_v[j, sl]
                w16 = ew_v[j, sl]
                a1 = plsc.load_gather(asrc_v, [s16])
                a2 = plsc.load_gather(adst_v, [d16])
                al = a1 + a2 + c_sc * w16
                al = jnp.maximum(al, 0.2 * al)
                ex16 = jnp.exp(al)
                for l in range(L):
                    ei = v * L + l
                    x = ex16[l]
                    for i in range(nscale):
                        sli = pl.ds(i * L, L)
                        bscat[ei, sli] = graw[ei, sli] * x
            pltpu.async_copy(bscat, acc.at[idx_d], ssem, add=True).wait()

        def ring(it, carry):
            j = it * 2
            pltpu.async_copy(hpad_hbm.at[src_v.at[j + 1]], graw1, gsem1)
            chunk(j, graw0, gsem0)
            pltpu.async_copy(hpad_hbm.at[src_v.at[j + 2]], graw0, gsem0)
            chunk(j + 1, graw1, gsem1)
            return carry

        lax.fori_loop(0, (nch - 1) // 2, ring, None)
        chunk(nch - 1, graw0, gsem0)
        plsc.subcore_barrier()
        pltpu.sync_copy(acc.at[pl.ds(s_ax * npw, npw)], out_hbm.at[c_ax, s_ax])

    def run(src_r, dst_r, ew_r, asrc_aug, adst, hpad):
        parts = sc_kernel(src_r, dst_r, ew_r, asrc_aug, adst, hpad)
        return parts.reshape(NC, n, p)

    return run


def _softmax(x, axis):
    m = jnp.max(x, axis=axis, keepdims=True)
    ex = jnp.exp(x - m)
    return ex / jnp.sum(ex, axis=axis, keepdims=True)


def _gru_body(x_ref, wat_ref, wih_ref, whh_ref, h_ref, *, t_steps, f_in):
    x = x_ref[...]
    wat = wat_ref[...]
    wih = wih_ref[...]
    whh = whh_ref[...]
    h = jnp.zeros((x.shape[0], H), jnp.float32)
    for t in range(t_steps):
        attn = _softmax(jnp.dot(h, wat), axis=0)
        xt = x[:, t * f_in:(t + 1) * f_in]
        gates = jnp.dot(xt * attn, wih) + jnp.dot(h, whh)
        r = jax.nn.sigmoid(gates[:, :H])
        u = jax.nn.sigmoid(gates[:, H:])
        h = u * h + (1.0 - u) * jnp.tanh(r * h)
    h_ref[...] = h


def _proj(h, att):
    return jnp.dot(h, att[0][:, None])


def _conv1_pre_body(xg_ref, w_ref, asr_ref, ads_ref, we_ref, ae_ref,
                    h_out_ref, asrc_ref, adst_ref, cvec_ref):
    xg = xg_ref[...]
    h = jnp.dot(xg, w_ref[...])
    c = jnp.sum(we_ref[...][0] * ae_ref[...][0])
    h_out_ref[...] = h
    asrc_ref[...] = _proj(h, asr_ref[...])
    adst_ref[...] = _proj(h, ads_ref[...])
    cvec_ref[...] = jnp.full((L,), c, jnp.float32)


def _conv2_pre_body(parts_ref, h1g_ref, asrc1_ref, asrc1_blk_ref, adst1_ref,
                    b1_ref, w2_ref, asr2_ref, ads2_ref, we2_ref, ae2_ref,
                    g1_ref, asrc2_ref, adst2_ref, exl2_ref,
                    cnt_ref, sumw_ref, cvec2_ref, *, n_total):
    ps = parts_ref[...]
    tot = ps[0] + ps[1]
    h1g = h1g_ref[...]
    asrc1f = asrc1_ref[...]
    c1 = asrc1f[n_total]
    asrc1 = asrc1_blk_ref[...][:, 0]
    adst1 = adst1_ref[...][:, 0]
    cnt = tot[:, 65]
    sumw = tot[:, 66]
    loop_attr = sumw / jnp.maximum(cnt, 1.0)
    al = asrc1 + adst1 + c1 * loop_attr
    al = jnp.maximum(al, 0.2 * al)
    exl1 = jnp.exp(al)
    num = tot[:, :64] + exl1[:, None] * h1g
    den = tot[:, 64] + exl1
    g1 = jnp.maximum(num / (den[:, None] + 1e-16) + b1_ref[...][None, :], 0.0)
    h2 = jnp.dot(g1, w2_ref[...])
    c2 = jnp.sum(we2_ref[...][0] * ae2_ref[...][0])
    asrc2 = _proj(h2, asr2_ref[...])
    adst2 = _proj(h2, ads2_ref[...])
    g1_ref[...] = g1
    asrc2_ref[...] = asrc2
    adst2_ref[...] = adst2
    al2 = asrc2[:, 0] + adst2[:, 0] + c2 * loop_attr
    al2 = jnp.maximum(al2, 0.2 * al2)
    exl2_ref[...] = jnp.exp(al2)[:, None]
    cnt_ref[...] = cnt[:, None]
    sumw_ref[...] = sumw[:, None]
    cvec2_ref[...] = jnp.full((L,), c2, jnp.float32)


def _mid_body(parts_ref, g1_ref, exl2_ref, w2_ref, b2_ref, h1_ref,
              mhs1_ref, mhs2_ref, wq_ref, wk_ref, wv_ref, bq_ref, bk_ref,
              bv_ref, wvsa_ref, bvsa_ref, wf1_ref, asr3_ref, ads3_ref,
              we3_ref, ae3_ref, cnt_ref, sumw_ref,
              hf1_ref, asrc3_ref, adst3_ref, exl3_ref, cvec3_ref):
    ps = parts_ref[...]
    tot = ps[0] + ps[1]
    g1 = g1_ref[...]
    exl2 = exl2_ref[...][:, 0]
    num = tot[:, :64] + exl2[:, None] * g1
    den = tot[:, 64] + exl2
    aggpre = num / (den[:, None] + 1e-16)
    g = jnp.dot(aggpre, w2_ref[...]) + b2_ref[...][None, :]

    h1 = h1_ref[...]
    wq = wq_ref[...]
    wk = wk_ref[...]
    wv = wv_ref[...]
    bq = bq_ref[...]
    bk = bk_ref[...]
    bv = bv_ref[...]
    sc = H ** -0.5

    def crossrep(q_in, kv):
        q = jnp.dot(q_in, wq) + bq[None, :]
        k = jnp.dot(kv, wk) + bk[None, :]
        v = jnp.dot(kv, wv) + bv[None, :]
        aw = _softmax(jnp.dot(q, k.T) * sc, axis=1)
        return jnp.dot(aw, v)

    rep1 = crossrep(h1, mhs1_ref[...])
    rep2 = crossrep(g, mhs2_ref[...])

    wvsa = wvsa_ref[...]
    wf1 = wf1_ref[...]
    wc0 = jnp.dot(wvsa[0 * H:1 * H], wf1)
    wc1 = jnp.dot(wvsa[1 * H:2 * H], wf1)
    wc2 = jnp.dot(wvsa[2 * H:3 * H], wf1)
    wc3 = jnp.dot(wvsa[3 * H:4 * H], wf1)
    bc = jnp.dot(bvsa_ref[...][None, :], wf1)[0]
    hf1 = (jnp.dot(h1, wc0) + jnp.dot(g, wc1) + jnp.dot(rep1, wc2)
           + jnp.dot(rep2, wc3) + bc[None, :])

    c3 = jnp.sum(we3_ref[...][0] * ae3_ref[...][0])
    hf1_ref[...] = hf1
    asrc3 = _proj(hf1, asr3_ref[...])
    adst3 = _proj(hf1, ads3_ref[...])
    asrc3_ref[...] = asrc3
    adst3_ref[...] = adst3
    cvec3_ref[...] = jnp.full((L,), c3, jnp.float32)
    loop_attr = sumw_ref[...][:, 0] / jnp.maximum(cnt_ref[...][:, 0], 1.0)
    al3 = asrc3[:, 0] + adst3[:, 0] + c3 * loop_attr
    al3 = jnp.maximum(al3, 0.2 * al3)
    exl3_ref[...] = jnp.exp(al3)[:, None]


def _conv4_pre_body(parts_ref, hf1_ref, exl3_ref, b3_ref, w4_ref,
                    asr4_ref, ads4_ref, we4_ref, ae4_ref, cnt_ref, sumw_ref,
                    hpad4_ref, asrc4_ref, adst4_ref, exl4_ref, cvec4_ref):
    ps = parts_ref[...]
    tot = ps[0] + ps[1]
    n = tot.shape[0]
    h3 = hf1_ref[...]
    exl3 = exl3_ref[...][:, 0]
    num = tot[:, :64] + exl3[:, None] * h3
    den = tot[:, 64] + exl3
    y1 = jnp.maximum(num / (den[:, None] + 1e-16) + b3_ref[...][None, :], 0.0)
    h4 = jnp.dot(y1, w4_ref[...])
    c4 = jnp.sum(we4_ref[...][0] * ae4_ref[...][0])
    hpad4_ref[...] = jnp.concatenate(
        [h4, jnp.ones((n, 1), jnp.float32), jnp.zeros((n, 14), jnp.float32)],
        axis=1)
    asrc4 = _proj(h4, asr4_ref[...])
    adst4 = _proj(h4, ads4_ref[...])
    asrc4_ref[...] = asrc4
    adst4_ref[...] = adst4
    cvec4_ref[...] = jnp.full((L,), c4, jnp.float32)
    loop_attr = sumw_ref[...][:, 0] / jnp.maximum(cnt_ref[...][:, 0], 1.0)
    al4 = asrc4[:, 0] + adst4[:, 0] + c4 * loop_attr
    al4 = jnp.maximum(al4, 0.2 * al4)
    exl4_ref[...] = jnp.exp(al4)[:, None]


def _final_body(parts_ref, hpad4_ref, exl4_ref, b4_ref, y_ref):
    ps = parts_ref[...]
    tot = ps[0] + ps[1]
    h4 = hpad4_ref[...][:, 0]
    exl4 = exl4_ref[...][:, 0]
    num = tot[:, 0] + exl4 * h4
    den = tot[:, 1] + exl4
    y = num / (den + 1e-16) + b4_ref[...][0]
    y_ref[...] = jnp.maximum(y, 0.0)[:, None]


def _tc_call(body, out_shapes, *args, **kw):
    return pl.pallas_call(
        functools.partial(body, **kw),
        out_shape=out_shapes,
    )(*args)


G = 10


def _tc_blocked(body, n, out_shapes, *args, **kw):
    nb = n // G

    def spec(shape):
        if n in shape:
            ax = shape.index(n)
            bshape = tuple(nb if i == ax else s for i, s in enumerate(shape))

            def im(i, ax=ax, r=len(shape)):
                return tuple(i if j == ax else 0 for j in range(r))

            return pl.BlockSpec(bshape, im)
        r = len(shape)
        return pl.BlockSpec(shape, lambda i, r=r: (0,) * r)

    in_specs = [spec(a.shape) for a in args]
    out_specs = jax.tree.map(lambda t: spec(t.shape), out_shapes)
    return pl.pallas_call(
        functools.partial(body, **kw),
        grid=(G,),
        in_specs=in_specs,
        out_specs=out_specs,
        out_shape=out_shapes,
    )(*args)


def kernel(x_time_series, x_graph, edge_index, edge_weight, params):
    n = x_graph.shape[0]
    e = edge_index.shape[1]
    t_steps = x_time_series.shape[2]
    f_in = x_time_series.shape[3]
    nch = e // (NW * K)

    src_r = edge_index[0].astype(jnp.int32).reshape(NW, nch, K)
    dst_r = edge_index[1].astype(jnp.int32).reshape(NW, nch, K)
    ew_r = edge_weight[:, 0].reshape(NW, nch, K)
    xflat = x_time_series[-1].reshape(n, t_steps * f_in)

    f32 = jnp.float32
    sd = jax.ShapeDtypeStruct

    gp = params["gru"]
    h1 = _tc_call(_gru_body, sd((n, H), f32),
                  xflat, gp["W_attn"], gp["W_ih"], gp["W_hh"],
                  t_steps=t_steps, f_in=f_in)

    g1p = params["gat1"]
    h1g, asrc1, adst1, cvec1 = _tc_blocked(
        _conv1_pre_body, n,
        (sd((n, 64), f32), sd((n, 1), f32), sd((n, 1), f32), sd((L,), f32)),
        x_graph, g1p["W"], g1p["att_src"], g1p["att_dst"], g1p["W_e"],
        g1p["att_edge"])
    asrc1_aug = jnp.concatenate([asrc1[:, 0], cvec1])

    sc_deg = _make_sc_wide_pass(n, e, with_deg=True)
    parts1 = sc_deg(src_r, dst_r, ew_r, asrc1_aug, adst1[:, 0], h1g)

    g2p = params["gat2"]
    (g1, asrc2, adst2, exl2, cnt, sumw, cvec2) = _tc_blocked(
        _conv2_pre_body, n,
        (sd((n, 64), f32), sd((n, 1), f32), sd((n, 1), f32), sd((n, 1), f32),
         sd((n, 1), f32), sd((n, 1), f32), sd((L,), f32)),
        parts1, h1g, asrc1_aug, asrc1, adst1, g1p["bias"],
        g2p["W"], g2p["att_src"], g2p["att_dst"], g2p["W_e"], g2p["att_edge"],
        n_total=n)
    asrc2_aug = jnp.concatenate([asrc2[:, 0], cvec2])

    sc_wide = _make_sc_wide_pass(n, e, with_deg=False)
    parts2 = sc_wide(src_r, dst_r, ew_r, asrc2_aug, adst2[:, 0], g1)

    ca = params["ca"]
    f1p = params["fgat1"]
    (hf1, asrc3, adst3, exl3, cvec3) = _tc_blocked(
        _mid_body, n,
        (sd((n, 64), f32), sd((n, 1), f32), sd((n, 1), f32), sd((n, 1), f32),
         sd((L,), f32)),
        parts2, g1, exl2, g2p["W"], g2p["bias"], h1,
        params["mhs1"], params["mhs2"], ca["Wq"], ca["Wk"], ca["Wv"],
        ca["bq"], ca["bk"], ca["bv"], params["sa"]["Wv"], params["sa"]["bv"],
        f1p["W"], f1p["att_src"], f1p["att_dst"], f1p["W_e"], f1p["att_edge"],
        cnt, sumw)
    asrc3_aug = jnp.concatenate([asrc3[:, 0], cvec3])

    parts3 = sc_wide(src_r, dst_r, ew_r, asrc3_aug, adst3[:, 0], hf1)

    f2p = params["fgat2"]
    (hpad4, asrc4, adst4, exl4, cvec4) = _tc_blocked(
        _conv4_pre_body, n,
        (sd((n, L), f32), sd((n, 1), f32), sd((n, 1), f32), sd((n, 1), f32),
         sd((L,), f32)),
        parts3, hf1, exl3, f1p["bias"],
        f2p["W"], f2p["att_src"], f2p["att_dst"], f2p["W_e"], f2p["att_edge"],
        cnt, sumw)
    asrc4_aug = jnp.concatenate([asrc4[:, 0], cvec4])

    sc4 = _make_sc_narrow_pass(n, e)
    parts4 = sc4(src_r, dst_r, ew_r, asrc4_aug, adst4[:, 0], hpad4)

    y = _tc_blocked(_final_body, n, sd((n, 1), f32),
                    parts4, hpad4, exl4, f2p["bias"])
    return y[:, 0]


# speedup vs baseline: 74.2915x; 1.0591x over previous
import functools

import jax
import jax.numpy as jnp
from jax import lax
from jax.experimental import pallas as pl
from jax.experimental.pallas import tpu as pltpu
from jax.experimental.pallas import tpu_sc as plsc

H = 128
NC, NS, L = 2, 16, 16
NW = NC * NS
K = 80


def _make_sc_wide_pass(n, e, with_deg):
    dg = 64
    p = dg + L
    epw = e // NW
    nch = epw // K
    npw = n // NS
    ZR = 25
    nzc = npw // ZR
    mesh = plsc.VectorSubcoreMesh(core_axis_name="c", subcore_axis_name="s")

    @functools.partial(
        pl.kernel,
        out_type=jax.ShapeDtypeStruct((NC, NS, n // NS, p), jnp.float32),
        mesh=mesh,
        compiler_params=pltpu.CompilerParams(needs_layout_passes=False,
                                             use_tc_tiling_on_sc=False),
        scratch_types=[
            pltpu.VMEM((nch, K), jnp.int32),
            pltpu.VMEM((nch, K), jnp.int32),
            pltpu.VMEM((nch, K), jnp.float32),
            pltpu.VMEM((n + L,), jnp.float32),
            pltpu.VMEM((n,), jnp.float32),
            pltpu.VMEM((K, dg), jnp.float32),
            pltpu.VMEM((K, dg), jnp.float32),
            pltpu.VMEM((K, p), jnp.float32),
            pltpu.VMEM((K, p), jnp.float32),
            pltpu.VMEM((ZR, p), jnp.float32),
            pltpu.VMEM_SHARED((n, p), jnp.float32),
            pltpu.SemaphoreType.DMA,
            pltpu.SemaphoreType.DMA,
            pltpu.SemaphoreType.DMA,
            pltpu.SemaphoreType.DMA,
        ],
    )
    def sc_kernel(src_hbm, dst_hbm, ew_hbm, asrc_hbm, adst_hbm, h_hbm,
                  out_hbm, src_v, dst_v, ew_v, asrc_v, adst_v, graw0,
                  graw1, bscat0, bscat1, zbuf, acc, gsem0, gsem1,
                  ssem0, ssem1):
        c_ax = lax.axis_index("c")
        s_ax = lax.axis_index("s")
        wid = c_ax * NS + s_ax
        pltpu.sync_copy(src_hbm.at[wid], src_v)
        pltpu.sync_copy(dst_hbm.at[wid], dst_v)
        pltpu.sync_copy(ew_hbm.at[wid], ew_v)
        pltpu.sync_copy(asrc_hbm, asrc_v)
        pltpu.sync_copy(adst_hbm, adst_v)

        z16 = jnp.zeros((L,), jnp.float32)

        def zrow(r, carry):
            for i in range(p // L):
                zbuf[r, pl.ds(i * L, L)] = z16
            return carry

        lax.fori_loop(0, ZR, zrow, None)

        def zcp(q, carry):
            pltpu.sync_copy(zbuf, acc.at[pl.ds(s_ax * npw + q * ZR, ZR)])
            return carry

        lax.fori_loop(0, nzc, zcp, None)
        plsc.subcore_barrier()

        c_sc = asrc_v[pl.ds(n, L)][0]
        lane = lax.iota(jnp.int32, L)
        m0 = lane == 0
        m1 = lane == 1
        m2 = lane == 2

        pltpu.async_copy(h_hbm.at[src_v.at[0]], graw0, gsem0)
        pltpu.async_copy(h_hbm.at[src_v.at[1]], graw1, gsem1)

        def chunk(j, graw, gsem, bsc, ssem):
            pltpu.make_async_copy(h_hbm.at[src_v.at[j]], graw, gsem).wait()
            for v in range(K // L):
                sl = pl.ds(v * L, L)
                s16 = src_v[j, sl]
                d16 = dst_v[j, sl]
                w16 = ew_v[j, sl]
                a1 = plsc.load_gather(asrc_v, [s16])
                a2 = plsc.load_gather(adst_v, [d16])
                al = a1 + a2 + c_sc * w16
                al = jnp.maximum(al, 0.2 * al)
                ex16 = jnp.exp(al)
                for l in range(L):
                    ei = v * L + l
                    x = ex16[l]
                    for i in range(dg // L):
                        sli = pl.ds(i * L, L)
                        bsc[ei, sli] = graw[ei, sli] * x
                    if with_deg:
                        aux = jnp.where(
                            m0, x, jnp.where(
                                m1, 1.0, jnp.where(m2, w16[l], 0.0)))
                    else:
                        aux = jnp.where(m0, x, 0.0)
                    bsc[ei, pl.ds(dg, L)] = aux
            pltpu.async_copy(bsc, acc.at[dst_v.at[j]], ssem, add=True)

        def sdrain(bsc, ssem):
            pltpu.make_async_copy(bsc, acc.at[dst_v.at[0]], ssem).wait()

        chunk(0, graw0, gsem0, bscat0, ssem0)
        pltpu.async_copy(h_hbm.at[src_v.at[2]], graw0, gsem0)

        def ring(it, carry):
            j = it * 2 + 1
            chunk(j, graw1, gsem1, bscat1, ssem1)
            sdrain(bscat0, ssem0)
            pltpu.async_copy(h_hbm.at[src_v.at[jnp.minimum(j + 2, nch - 1)]],
                             graw1, gsem1)
            chunk(j + 1, graw0, gsem0, bscat0, ssem0)
            sdrain(bscat1, ssem1)
            pltpu.async_copy(h_hbm.at[src_v.at[jnp.minimum(j + 3, nch - 1)]],
                             graw0, gsem0)
            return carry

        lax.fori_loop(0, (nch - 1) // 2, ring, None)
        sdrain(bscat0, ssem0)
        pltpu.make_async_copy(h_hbm.at[src_v.at[0]], graw0, gsem0).wait()
        pltpu.make_async_copy(h_hbm.at[src_v.at[0]], graw1, gsem1).wait()
        plsc.subcore_barrier()
        pltpu.sync_copy(acc.at[pl.ds(s_ax * npw, npw)], out_hbm.at[c_ax, s_ax])

    def run(src_r, dst_r, ew_r, asrc_aug, adst, h):
        parts = sc_kernel(src_r, dst_r, ew_r, asrc_aug, adst, h)
        return parts.reshape(NC, n, p)

    return run


def _make_sc_narrow_pass(n, e):
    p = L
    d = 1
    epw = e // NW
    nch = epw // K
    npw = n // NS
    ZR = 25
    nzc = npw // ZR
    nscale = (d + 1 + (L - 1)) // L
    mesh = plsc.VectorSubcoreMesh(core_axis_name="c", subcore_axis_name="s")

    @functools.partial(
        pl.kernel,
        out_type=jax.ShapeDtypeStruct((NC, NS, n // NS, p), jnp.float32),
        mesh=mesh,
        compiler_params=pltpu.CompilerParams(needs_layout_passes=False,
                                             use_tc_tiling_on_sc=False),
        scratch_types=[
            pltpu.VMEM((nch, K), jnp.int32),
            pltpu.VMEM((nch, K), jnp.int32),
            pltpu.VMEM((nch, K), jnp.float32),
            pltpu.VMEM((n + L,), jnp.float32),
            pltpu.VMEM((n,), jnp.float32),
            pltpu.VMEM((K, p), jnp.float32),
            pltpu.VMEM((K, p), jnp.float32),
            pltpu.VMEM((K, p), jnp.float32),
            pltpu.VMEM((K, p), jnp.float32),
            pltpu.VMEM((ZR, p), jnp.float32),
            pltpu.VMEM_SHARED((n, p), jnp.float32),
            pltpu.SemaphoreType.DMA,
            pltpu.SemaphoreType.DMA,
            pltpu.SemaphoreType.DMA,
            pltpu.SemaphoreType.DMA,
        ],
    )
    def sc_kernel(src_hbm, dst_hbm, ew_hbm, asrc_hbm, adst_hbm, hpad_hbm,
                  out_hbm, src_v, dst_v, ew_v, asrc_v, adst_v, graw0,
                  graw1, bscat0, bscat1, zbuf, acc, gsem0, gsem1,
                  ssem0, ssem1):
        c_ax = lax.axis_index("c")
        s_ax = lax.axis_index("s")
        wid = c_ax * NS + s_ax
        pltpu.sync_copy(src_hbm.at[wid], src_v)
        pltpu.sync_copy(dst_hbm.at[wid], dst_v)
        pltpu.sync_copy(ew_hbm.at[wid], ew_v)
        pltpu.sync_copy(asrc_hbm, asrc_v)
        pltpu.sync_copy(adst_hbm, adst_v)

        z16 = jnp.zeros((L,), jnp.float32)

        def zrow(r, carry):
            zbuf[r, pl.ds(0, L)] = z16
            return carry

        lax.fori_loop(0, ZR, zrow, None)

        def zcp(q, carry):
            pltpu.sync_copy(zbuf, acc.at[pl.ds(s_ax * npw + q * ZR, ZR)])
            return carry

        lax.fori_loop(0, nzc, zcp, None)
        plsc.subcore_barrier()

        c_sc = asrc_v[pl.ds(n, L)][0]

        pltpu.async_copy(hpad_hbm.at[src_v.at[0]], graw0, gsem0)
        pltpu.async_copy(hpad_hbm.at[src_v.at[1]], graw1, gsem1)

        def chunk(j, graw, gsem, bsc, ssem):
            pltpu.make_async_copy(hpad_hbm.at[src_v.at[j]], graw, gsem).wait()
            for v in range(K // L):
                sl = pl.ds(v * L, L)
                s16 = src_v[j, sl]
                d16 = dst_v[j, sl]
                w16 = ew_v[j, sl]
                a1 = plsc.load_gather(asrc_v, [s16])
                a2 = plsc.load_gather(adst_v, [d16])
                al = a1 + a2 + c_sc * w16
                al = jnp.maximum(al, 0.2 * al)
                ex16 = jnp.exp(al)
                for l in range(L):
                    ei = v * L + l
                    x = ex16[l]
                    for i in range(nscale):
                        sli = pl.ds(i * L, L)
                        bsc[ei, sli] = graw[ei, sli] * x
            pltpu.async_copy(bsc, acc.at[dst_v.at[j]], ssem, add=True)

        def sdrain(bsc, ssem):
            pltpu.make_async_copy(bsc, acc.at[dst_v.at[0]], ssem).wait()

        chunk(0, graw0, gsem0, bscat0, ssem0)
        pltpu.async_copy(hpad_hbm.at[src_v.at[2]], graw0, gsem0)

        def ring(it, carry):
            j = it * 2 + 1
            chunk(j, graw1, gsem1, bscat1, ssem1)
            sdrain(bscat0, ssem0)
            pltpu.async_copy(
                hpad_hbm.at[src_v.at[jnp.minimum(j + 2, nch - 1)]],
                graw1, gsem1)
            chunk(j + 1, graw0, gsem0, bscat0, ssem0)
            sdrain(bscat1, ssem1)
            pltpu.async_copy(
                hpad_hbm.at[src_v.at[jnp.minimum(j + 3, nch - 1)]],
                graw0, gsem0)
            return carry

        lax.fori_loop(0, (nch - 1) // 2, ring, None)
        sdrain(bscat0, ssem0)
        pltpu.make_async_copy(hpad_hbm.at[src_v.at[0]], graw0, gsem0).wait()
        pltpu.make_async_copy(hpad_hbm.at[src_v.at[0]], graw1, gsem1).wait()
        plsc.subcore_barrier()
        pltpu.sync_copy(acc.at[pl.ds(s_ax * npw, npw)], out_hbm.at[c_ax, s_ax])

    def run(src_r, dst_r, ew_r, asrc_aug, adst, hpad):
        parts = sc_kernel(src_r, dst_r, ew_r, asrc_aug, adst, hpad)
        return parts.reshape(NC, n, p)

    return run


def _softmax(x, axis):
    m = jnp.max(x, axis=axis, keepdims=True)
    ex = jnp.exp(x - m)
    return ex / jnp.sum(ex, axis=axis, keepdims=True)


def _gru_body(x_ref, wat_ref, wih_ref, whh_ref, h_ref, *, t_steps, f_in):
    x = x_ref[...]
    wat = wat_ref[...]
    wih = wih_ref[...]
    whh = whh_ref[...]
    h = jnp.zeros((x.shape[0], H), jnp.float32)
    for t in range(t_steps):
        attn = _softmax(jnp.dot(h, wat), axis=0)
        xt = x[:, t * f_in:(t + 1) * f_in]
        gates = jnp.dot(xt * attn, wih) + jnp.dot(h, whh)
        r = jax.nn.sigmoid(gates[:, :H])
        u = jax.nn.sigmoid(gates[:, H:])
        h = u * h + (1.0 - u) * jnp.tanh(r * h)
    h_ref[...] = h


def _proj(h, att):
    return jnp.dot(h, att[0][:, None])


def _conv1_pre_body(xg_ref, w_ref, asr_ref, ads_ref, we_ref, ae_ref,
                    h_out_ref, asrc_ref, adst_ref, cvec_ref):
    xg = xg_ref[...]
    h = jnp.dot(xg, w_ref[...])
    c = jnp.sum(we_ref[...][0] * ae_ref[...][0])
    h_out_ref[...] = h
    asrc_ref[...] = _proj(h, asr_ref[...])
    adst_ref[...] = _proj(h, ads_ref[...])
    cvec_ref[...] = jnp.full((L,), c, jnp.float32)


def _conv2_pre_body(parts_ref, h1g_ref, asrc1_ref, asrc1_blk_ref, adst1_ref,
                    b1_ref, w2_ref, asr2_ref, ads2_ref, we2_ref, ae2_ref,
                    g1_ref, asrc2_ref, adst2_ref, exl2_ref,
                    cnt_ref, sumw_ref, cvec2_ref, *, n_total):
    ps = parts_ref[...]
    tot = ps[0] + ps[1]
    h1g = h1g_ref[...]
    asrc1f = asrc1_ref[...]
    c1 = asrc1f[n_total]
    asrc1 = asrc1_blk_ref[...][:, 0]
    adst1 = adst1_ref[...][:, 0]
    cnt = tot[:, 65]
    sumw = tot[:, 66]
    loop_attr = sumw / jnp.maximum(cnt, 1.0)
    al = asrc1 + adst1 + c1 * loop_attr
    al = jnp.maximum(al, 0.2 * al)
    exl1 = jnp.exp(al)
    num = tot[:, :64] + exl1[:, None] * h1g
    den = tot[:, 64] + exl1
    g1 = jnp.maximum(num / (den[:, None] + 1e-16) + b1_ref[...][None, :], 0.0)
    h2 = jnp.dot(g1, w2_ref[...])
    c2 = jnp.sum(we2_ref[...][0] * ae2_ref[...][0])
    asrc2 = _proj(h2, asr2_ref[...])
    adst2 = _proj(h2, ads2_ref[...])
    g1_ref[...] = g1
    asrc2_ref[...] = asrc2
    adst2_ref[...] = adst2
    al2 = asrc2[:, 0] + adst2[:, 0] + c2 * loop_attr
    al2 = jnp.maximum(al2, 0.2 * al2)
    exl2_ref[...] = jnp.exp(al2)[:, None]
    cnt_ref[...] = cnt[:, None]
    sumw_ref[...] = sumw[:, None]
    cvec2_ref[...] = jnp.full((L,), c2, jnp.float32)


def _mid_body(parts_ref, g1_ref, exl2_ref, w2_ref, b2_ref, h1_ref,
              mhs1_ref, mhs2_ref, wq_ref, wk_ref, wv_ref, bq_ref, bk_ref,
              bv_ref, wvsa_ref, bvsa_ref, wf1_ref, asr3_ref, ads3_ref,
              we3_ref, ae3_ref, cnt_ref, sumw_ref,
              hf1_ref, asrc3_ref, adst3_ref, exl3_ref, cvec3_ref):
    ps = parts_ref[...]
    tot = ps[0] + ps[1]
    g1 = g1_ref[...]
    exl2 = exl2_ref[...][:, 0]
    num = tot[:, :64] + exl2[:, None] * g1
    den = tot[:, 64] + exl2
    aggpre = num / (den[:, None] + 1e-16)
    g = jnp.dot(aggpre, w2_ref[...]) + b2_ref[...][None, :]

    h1 = h1_ref[...]
    wq = wq_ref[...]
    wk = wk_ref[...]
    wv = wv_ref[...]
    bq = bq_ref[...]
    bk = bk_ref[...]
    bv = bv_ref[...]
    sc = H ** -0.5

    def crossrep(q_in, kv):
        q = jnp.dot(q_in, wq) + bq[None, :]
        k = jnp.dot(kv, wk) + bk[None, :]
        v = jnp.dot(kv, wv) + bv[None, :]
        aw = _softmax(jnp.dot(q, k.T) * sc, axis=1)
        return jnp.dot(aw, v)

    rep1 = crossrep(h1, mhs1_ref[...])
    rep2 = crossrep(g, mhs2_ref[...])

    wvsa = wvsa_ref[...]
    wf1 = wf1_ref[...]
    wc0 = jnp.dot(wvsa[0 * H:1 * H], wf1)
    wc1 = jnp.dot(wvsa[1 * H:2 * H], wf1)
    wc2 = jnp.dot(wvsa[2 * H:3 * H], wf1)
    wc3 = jnp.dot(wvsa[3 * H:4 * H], wf1)
    bc = jnp.dot(bvsa_ref[...][None, :], wf1)[0]
    hf1 = (jnp.dot(h1, wc0) + jnp.dot(g, wc1) + jnp.dot(rep1, wc2)
           + jnp.dot(rep2, wc3) + bc[None, :])

    c3 = jnp.sum(we3_ref[...][0] * ae3_ref[...][0])
    hf1_ref[...] = hf1
    asrc3 = _proj(hf1, asr3_ref[...])
    adst3 = _proj(hf1, ads3_ref[...])
    asrc3_ref[...] = asrc3
    adst3_ref[...] = adst3
    cvec3_ref[...] = jnp.full((L,), c3, jnp.float32)
    loop_attr = sumw_ref[...][:, 0] / jnp.maximum(cnt_ref[...][:, 0], 1.0)
    al3 = asrc3[:, 0] + adst3[:, 0] + c3 * loop_attr
    al3 = jnp.maximum(al3, 0.2 * al3)
    exl3_ref[...] = jnp.exp(al3)[:, None]


def _conv4_pre_body(parts_ref, hf1_ref, exl3_ref, b3_ref, w4_ref,
                    asr4_ref, ads4_ref, we4_ref, ae4_ref, cnt_ref, sumw_ref,
                    hpad4_ref, asrc4_ref, adst4_ref, exl4_ref, cvec4_ref):
    ps = parts_ref[...]
    tot = ps[0] + ps[1]
    n = tot.shape[0]
    h3 = hf1_ref[...]
    exl3 = exl3_ref[...][:, 0]
    num = tot[:, :64] + exl3[:, None] * h3
    den = tot[:, 64] + exl3
    y1 = jnp.maximum(num / (den[:, None] + 1e-16) + b3_ref[...][None, :], 0.0)
    h4 = jnp.dot(y1, w4_ref[...])
    c4 = jnp.sum(we4_ref[...][0] * ae4_ref[...][0])
    hpad4_ref[...] = jnp.concatenate(
        [h4, jnp.ones((n, 1), jnp.float32), jnp.zeros((n, 14), jnp.float32)],
        axis=1)
    asrc4 = _proj(h4, asr4_ref[...])
    adst4 = _proj(h4, ads4_ref[...])
    asrc4_ref[...] = asrc4
    adst4_ref[...] = adst4
    cvec4_ref[...] = jnp.full((L,), c4, jnp.float32)
    loop_attr = sumw_ref[...][:, 0] / jnp.maximum(cnt_ref[...][:, 0], 1.0)
    al4 = asrc4[:, 0] + adst4[:, 0] + c4 * loop_attr
    al4 = jnp.maximum(al4, 0.2 * al4)
    exl4_ref[...] = jnp.exp(al4)[:, None]


def _final_body(parts_ref, hpad4_ref, exl4_ref, b4_ref, y_ref):
    ps = parts_ref[...]
    tot = ps[0] + ps[1]
    h4 = hpad4_ref[...][:, 0]
    exl4 = exl4_ref[...][:, 0]
    num = tot[:, 0] + exl4 * h4
    den = tot[:, 1] + exl4
    y = num / (den + 1e-16) + b4_ref[...][0]
    y_ref[...] = jnp.maximum(y, 0.0)[:, None]


def _tc_call(body, out_shapes, *args, **kw):
    return pl.pallas_call(
        functools.partial(body, **kw),
        out_shape=out_shapes,
    )(*args)


G = 10


def _tc_blocked(body, n, out_shapes, *args, **kw):
    nb = n // G

    def spec(shape):
        if n in shape:
            ax = shape.index(n)
            bshape = tuple(nb if i == ax else s for i, s in enumerate(shape))

            def im(i, ax=ax, r=len(shape)):
                return tuple(i if j == ax else 0 for j in range(r))

            return pl.BlockSpec(bshape, im)
        r = len(shape)
        return pl.BlockSpec(shape, lambda i, r=r: (0,) * r)

    in_specs = [spec(a.shape) for a in args]
    out_specs = jax.tree.map(lambda t: spec(t.shape), out_shapes)
    return pl.pallas_call(
        functools.partial(body, **kw),
        grid=(G,),
        in_specs=in_specs,
        out_specs=out_specs,
        out_shape=out_shapes,
    )(*args)


def kernel(x_time_series, x_graph, edge_index, edge_weight, params):
    n = x_graph.shape[0]
    e = edge_index.shape[1]
    t_steps = x_time_series.shape[2]
    f_in = x_time_series.shape[3]
    nch = e // (NW * K)

    src_r = edge_index[0].astype(jnp.int32).reshape(NW, nch, K)
    dst_r = edge_index[1].astype(jnp.int32).reshape(NW, nch, K)
    ew_r = edge_weight[:, 0].reshape(NW, nch, K)
    xflat = x_time_series[-1].reshape(n, t_steps * f_in)

    f32 = jnp.float32
    sd = jax.ShapeDtypeStruct

    gp = params["gru"]
    h1 = _tc_call(_gru_body, sd((n, H), f32),
                  xflat, gp["W_attn"], gp["W_ih"], gp["W_hh"],
                  t_steps=t_steps, f_in=f_in)

    g1p = params["gat1"]
    h1g, asrc1, adst1, cvec1 = _tc_blocked(
        _conv1_pre_body, n,
        (sd((n, 64), f32), sd((n, 1), f32), sd((n, 1), f32), sd((L,), f32)),
        x_graph, g1p["W"], g1p["att_src"], g1p["att_dst"], g1p["W_e"],
        g1p["att_edge"])
    asrc1_aug = jnp.concatenate([asrc1[:, 0], cvec1])

    sc_deg = _make_sc_wide_pass(n, e, with_deg=True)
    parts1 = sc_deg(src_r, dst_r, ew_r, asrc1_aug, adst1[:, 0], h1g)

    g2p = params["gat2"]
    (g1, asrc2, adst2, exl2, cnt, sumw, cvec2) = _tc_blocked(
        _conv2_pre_body, n,
        (sd((n, 64), f32), sd((n, 1), f32), sd((n, 1), f32), sd((n, 1), f32),
         sd((n, 1), f32), sd((n, 1), f32), sd((L,), f32)),
        parts1, h1g, asrc1_aug, asrc1, adst1, g1p["bias"],
        g2p["W"], g2p["att_src"], g2p["att_dst"], g2p["W_e"], g2p["att_edge"],
        n_total=n)
    asrc2_aug = jnp.concatenate([asrc2[:, 0], cvec2])

    sc_wide = _make_sc_wide_pass(n, e, with_deg=False)
    parts2 = sc_wide(src_r, dst_r, ew_r, asrc2_aug, adst2[:, 0], g1)

    ca = params["ca"]
    f1p = params["fgat1"]
    (hf1, asrc3, adst3, exl3, cvec3) = _tc_blocked(
        _mid_body, n,
        (sd((n, 64), f32), sd((n, 1), f32), sd((n, 1), f32), sd((n, 1), f32),
         sd((L,), f32)),
        parts2, g1, exl2, g2p["W"], g2p["bias"], h1,
        params["mhs1"], params["mhs2"], ca["Wq"], ca["Wk"], ca["Wv"],
        ca["bq"], ca["bk"], ca["bv"], params["sa"]["Wv"], params["sa"]["bv"],
        f1p["W"], f1p["att_src"], f1p["att_dst"], f1p["W_e"], f1p["att_edge"],
        cnt, sumw)
    asrc3_aug = jnp.concatenate([asrc3[:, 0], cvec3])

    parts3 = sc_wide(src_r, dst_r, ew_r, asrc3_aug, adst3[:, 0], hf1)

    f2p = params["fgat2"]
    (hpad4, asrc4, adst4, exl4, cvec4) = _tc_blocked(
        _conv4_pre_body, n,
        (sd((n, L), f32), sd((n, 1), f32), sd((n, 1), f32), sd((n, 1), f32),
         sd((L,), f32)),
        parts3, hf1, exl3, f1p["bias"],
        f2p["W"], f2p["att_src"], f2p["att_dst"], f2p["W_e"], f2p["att_edge"],
        cnt, sumw)
    asrc4_aug = jnp.concatenate([asrc4[:, 0], cvec4])

    sc4 = _make_sc_narrow_pass(n, e)
    parts4 = sc4(src_r, dst_r, ew_r, asrc4_aug, adst4[:, 0], hpad4)

    y = _tc_blocked(_final_body, n, sd((n, 1), f32),
                    parts4, hpad4, exl4, f2p["bias"])
    return y[:, 0]
